# Initial kernel scaffold; baseline (speedup 1.0000x reference)
import jax
import jax.numpy as jnp
from jax.experimental import pallas as pl


def kernel(x, edge_index, W1, b1, W2, b2, W3, b3):
    raise NotImplementedError("write your pallas kernel here")


# speedup vs baseline: 16.7778x; 16.7778x over previous
import functools

import jax
import jax.numpy as jnp
from jax import lax
from jax.experimental import pallas as pl
from jax.experimental.pallas import tpu as pltpu
from jax.experimental.pallas import tpu_sc as plsc

NC = 2
NS = 16
NW = NC * NS
CHUNK = 64
RINIT = 128
DEGW = 128


def _mesh():
    return plsc.VectorSubcoreMesh(core_axis_name="c", subcore_axis_name="s")


@functools.lru_cache(maxsize=None)
def _deg_kernel(np_rows, nch, degw):
    @functools.partial(
        pl.kernel,
        mesh=_mesh(),
        out_type=jax.ShapeDtypeStruct((NC, np_rows, degw), jnp.float32),
        scratch_types=[
            pltpu.VMEM((nch, CHUNK), jnp.int32),
            pltpu.VMEM((CHUNK, degw), jnp.float32),
            pltpu.VMEM_SHARED((np_rows, degw), jnp.float32),
        ],
    )
    def kern(dst_hbm, ones_hbm, out, didx, ones_v, acc):
        c = lax.axis_index("c")
        s = lax.axis_index("s")
        wid = c * NS + s
        pltpu.sync_copy(ones_hbm, ones_v)
        pltpu.sync_copy(dst_hbm.at[pl.ds(wid * nch, nch)], didx)

        rpt = np_rows // NS
        r0 = s * rpt
        def init_body(k, _):
            pltpu.sync_copy(ones_v, acc.at[pl.ds(r0 + k * CHUNK, CHUNK)])
            return 0
        lax.fori_loop(0, rpt // CHUNK, init_body, 0)
        plsc.subcore_barrier()

        def body(j, _):
            pltpu.sync_copy(ones_v, acc.at[didx.at[j]], add=True)
            return 0
        lax.fori_loop(0, nch, body, 0)
        plsc.subcore_barrier()

        def wb_body(k, _):
            rb = r0 + k * CHUNK
            pltpu.sync_copy(acc.at[pl.ds(rb, CHUNK)], ones_v)
            pltpu.sync_copy(ones_v, out.at[c, pl.ds(rb, CHUNK)])
            return 0
        lax.fori_loop(0, rpt // CHUNK, wb_body, 0)

    return kern


@functools.lru_cache(maxsize=None)
def _prop_kernel(np_rows, d, nch):
    @functools.partial(
        pl.kernel,
        mesh=_mesh(),
        out_type=jax.ShapeDtypeStruct((NC, np_rows, d), jnp.float32),
        scratch_types=[
            pltpu.VMEM((nch // 2, CHUNK), jnp.int32),
            pltpu.VMEM((nch // 2, CHUNK), jnp.int32),
            pltpu.VMEM((CHUNK, d), jnp.float32),
            pltpu.VMEM((CHUNK, d), jnp.float32),
            pltpu.VMEM_SHARED((np_rows, d), jnp.float32),
            pltpu.SemaphoreType.DMA,
            pltpu.SemaphoreType.DMA,
        ],
    )
    def kern(table, src_hbm, dst_hbm, out, sidx, didx, rows0, rows1,
             acc, sem0, sem1):
        c = lax.axis_index("c")
        s = lax.axis_index("s")
        wid = c * NS + s

        rpt = np_rows // NS
        r0 = s * rpt
        def init_body(k, _):
            rb = r0 + k * CHUNK
            pltpu.sync_copy(table.at[pl.ds(rb, CHUNK)], rows0)
            pltpu.sync_copy(rows0, acc.at[pl.ds(rb, CHUNK)])
            return 0
        lax.fori_loop(0, rpt // CHUNK, init_body, 0)
        plsc.subcore_barrier()

        rows = (rows0, rows1)
        sems = (sem0, sem1)

        def gather(j, b):
            return pltpu.make_async_copy(table.at[sidx.at[j]], rows[b], sems[b])

        def scatter(j, b):
            pltpu.sync_copy(rows[b], acc.at[didx.at[j]], add=True)

        nchp = nch // 2
        nch2 = nchp // 2
        for h in range(2):
            base = wid * nch + h * nchp
            pltpu.sync_copy(src_hbm.at[pl.ds(base, nchp)], sidx)
            pltpu.sync_copy(dst_hbm.at[pl.ds(base, nchp)], didx)
            gather(0, 0).start()

            def body(i, _):
                j0 = 2 * i
                gather(j0 + 1, 1).start()
                gather(j0, 0).wait()
                scatter(j0, 0)

                @pl.when(i < nch2 - 1)
                def _():
                    gather(j0 + 2, 0).start()

                gather(j0 + 1, 1).wait()
                scatter(j0 + 1, 1)
                return 0

            lax.fori_loop(0, nch2, body, 0)
        plsc.subcore_barrier()

        def wb_body(k, _):
            rb = r0 + k * CHUNK
            pltpu.sync_copy(acc.at[pl.ds(rb, CHUNK)], rows0)
            pltpu.sync_copy(rows0, out.at[c, pl.ds(rb, CHUNK)])
            return 0
        lax.fori_loop(0, rpt // CHUNK, wb_body, 0)

    return kern


def _norm_rows(pdeg, x_pad):
    np_rows, d = x_pad.shape
    blk = 1280
    def body(pdeg_ref, x_ref, o_ref):
        deg = jnp.maximum(
            pdeg_ref[0, :, 0:1] + pdeg_ref[1, :, 0:1] - 1.0, 1.0)
        o_ref[...] = x_ref[...] * lax.rsqrt(deg)
    return pl.pallas_call(
        body,
        grid=(np_rows // blk,),
        in_specs=[
            pl.BlockSpec((NC, blk, DEGW), lambda i: (0, i, 0)),
            pl.BlockSpec((blk, d), lambda i: (i, 0)),
        ],
        out_specs=pl.BlockSpec((blk, d), lambda i: (i, 0)),
        out_shape=jax.ShapeDtypeStruct((np_rows, d), jnp.float32),
    )(pdeg, x_pad)


def _dense_layer(p, u, pdeg, w, b, *, n_out):
    np_rows, d = u.shape
    last = n_out is not None
    blk = 2000 if last else 1280
    grid = (n_out // blk,) if last else (np_rows // blk,)
    def body(p_ref, u_ref, pdeg_ref, w_ref, b_ref, o_ref):
        deg = jnp.maximum(
            pdeg_ref[0, :, 0:1] + pdeg_ref[1, :, 0:1] - 1.0, 1.0)
        nrm = lax.rsqrt(deg)
        a = p_ref[0] + p_ref[1] - u_ref[...]
        z = jnp.dot(a, w_ref[...], preferred_element_type=jnp.float32)
        if last:
            o_ref[...] = z * nrm + b_ref[...]
        else:
            o_ref[...] = jnp.maximum(z / deg + nrm * b_ref[...], 0.0)
    return pl.pallas_call(
        body,
        grid=grid,
        in_specs=[
            pl.BlockSpec((NC, blk, d), lambda i: (0, i, 0)),
            pl.BlockSpec((blk, d), lambda i: (i, 0)),
            pl.BlockSpec((NC, blk, DEGW), lambda i: (0, i, 0)),
            pl.BlockSpec((d, d), lambda i: (0, 0)),
            pl.BlockSpec((1, d), lambda i: (0, 0)),
        ],
        out_specs=pl.BlockSpec((blk, d), lambda i: (i, 0)),
        out_shape=jax.ShapeDtypeStruct(
            (n_out if last else np_rows, d), jnp.float32),
    )(p, u, pdeg, w, b.reshape(1, d))


def kernel(x, edge_index, W1, b1, W2, b2, W3, b3):
    n, d = x.shape
    e = edge_index.shape[1]

    np_rows = -(-n // (NS * RINIT)) * NS * RINIT
    nch = -(-e // (NW * CHUNK * 8)) * 8
    ep = NW * CHUNK * nch
    pad = ep - e

    idx = jnp.arange(pad, dtype=jnp.int32)
    src = jnp.concatenate([edge_index[0], idx % n])
    dst = jnp.concatenate([edge_index[1], n + (idx % DEGW)])
    src2 = src.reshape(NW * nch, CHUNK)
    dst2 = dst.reshape(NW * nch, CHUNK)
    x_pad = jnp.pad(x, ((0, np_rows - n), (0, 0)))

    ones = jnp.ones((CHUNK, DEGW), jnp.float32)
    pdeg = _deg_kernel(np_rows, nch, DEGW)(dst2, ones)

    prop = _prop_kernel(np_rows, d, nch)
    u0 = _norm_rows(pdeg, x_pad)
    p1 = prop(u0, src2, dst2)
    u1 = _dense_layer(p1, u0, pdeg, W1, b1, n_out=None)
    p2 = prop(u1, src2, dst2)
    u2 = _dense_layer(p2, u1, pdeg, W2, b2, n_out=None)
    p3 = prop(u2, src2, dst2)
    return _dense_layer(p3, u2, pdeg, W3, b3, n_out=n)
